# baseline (speedup 1.0000x reference)
import jax
import jax.numpy as jnp
from jax import lax
from jax.experimental import pallas as pl
from jax.experimental.pallas import tpu as pltpu
from jax.experimental.pallas import tpu_sc as plsc

_B = 128
_N = 32768
_OUT_DIM = 8192
_DPC = 4
_BOOST_STRENGTH = 1.0
_TARGET_DENSITY = float(_OUT_DIM) / _N

_NC = 2
_NS = 16
_NW = _NC * _NS
_GPW = _OUT_DIM // _NW
_WSPAN = (_DPC - 1) * _GPW
_WPAD = _WSPAN + 128
_XLEN = _DPC * _GPW
_R = 8
_NCHUNK = _B // _R
_NBLK = _GPW // 16

_SHW = 20480
_STRIPE = _SHW // _NS


def _dkw_body(x_hbm, duty_hbm, out_hbm,
              dv, bf_tile, xw0, xw1, xo0, xo1, ob0, ob1, sh0, sh1,
              sem_w0, sem_w1, sem_o0, sem_o1, sem_out0, sem_out1,
              sem_sp0, sem_sp1):
  cid = lax.axis_index("c")
  sid = lax.axis_index("s")
  wid = cid * _NS + sid
  wcol = wid * _WSPAN
  xcol = wid * _XLEN
  sc_lo = cid * 12288
  wloc = sid * _WSPAN
  xoloc = cid * 4096 + sid * _XLEN
  sploc = sid * _STRIPE

  iota = lax.iota(jnp.int32, 16)
  iota3 = iota * (_DPC - 1)

  xw = (xw0, xw1)
  xo = (xo0, xo1)
  ob = (ob0, ob1)
  sh = (sh0, sh1)
  sem_w = (sem_w0, sem_w1)
  sem_o = (sem_o0, sem_o1)
  sem_out = (sem_out0, sem_out1)
  sem_sp = (sem_sp0, sem_sp1)

  def stage_copy(c, s):
    rows = pl.ds(c * _R, _R)
    return pltpu.make_async_copy(
        x_hbm.at[rows, pl.ds(sc_lo + sploc, _STRIPE)],
        sh[s].at[:, pl.ds(sploc, _STRIPE)], sem_sp[s])

  def pull_copies(s):
    return [
        pltpu.make_async_copy(
            sh[s].at[:, pl.ds(wloc, _WPAD)], xw[s], sem_w[s]),
        pltpu.make_async_copy(
            sh[s].at[:, pl.ds(xoloc, _XLEN)], xo[s], sem_o[s]),
    ]

  def out_copies(c, s):
    rows = pl.ds(c * _R, _R)
    return [pltpu.make_async_copy(
        ob[s], out_hbm.at[rows, pl.ds(xcol, _XLEN)], sem_out[s])]

  dnums = lax.GatherDimensionNumbers(
      offset_dims=(), collapsed_slice_dims=(0,), start_index_map=(0,))

  def vgather(v, idx):
    return lax.gather(v, idx[:, None], dnums, (1,),
                      mode=lax.GatherScatterMode.PROMISE_IN_BOUNDS)

  perm_base = iota // 4
  slot = iota - perm_base * 4

  def compute(s):
    xw_s, xo_s, ob_s = xw[s], xo[s], ob[s]

    @plsc.parallel_loop(0, _NBLK)
    def _(blk):
      cw = iota3 + blk * 48
      ob_off = blk * 64
      bfv = [bf_tile[pl.ds(blk * 64 + j * 16, 16)] for j in range(_DPC)]

      @plsc.parallel_loop(0, _R, unroll=2)
      def _(r):
        rvec = jnp.full((16,), r, jnp.int32)
        m = plsc.load_gather(xw_s, [rvec, cw]) * bfv[0]
        ind = jnp.zeros((16,), jnp.int32)
        for j in range(1, _DPC):
          wj = plsc.load_gather(xw_s, [rvec, cw + j]) * bfv[j]
          gt = wj > m
          m = jnp.where(gt, wj, m)
          ind = jnp.where(gt, j, ind)
        z = jnp.zeros((16,), jnp.float32)
        for j in range(_DPC):
          indp = vgather(ind, perm_base + 4 * j)
          xov = xo_s[r, pl.ds(ob_off + j * 16, 16)]
          ob_s[r, pl.ds(ob_off + j * 16, 16)] = jnp.where(indp == slot, xov, z)

  def bf_prologue():
    pltpu.sync_copy(duty_hbm.at[pl.ds(wcol, _WPAD)], dv)

    @pl.loop(0, _WPAD // 16)
    def _(i):
      v = dv[pl.ds(i * 16, 16)]
      dv[pl.ds(i * 16, 16)] = jnp.exp((_TARGET_DENSITY - v) * _BOOST_STRENGTH)

    @pl.loop(0, _NBLK)
    def _(blk):
      base = blk * 48
      for j in range(_DPC):
        vals = plsc.load_gather(dv, [iota3 + (base + j)])
        bf_tile[pl.ds(blk * 64 + j * 16, 16)] = vals

  stage_copy(0, 0).start()
  stage_copy(1, 1).start()
  bf_prologue()
  stage_copy(0, 0).wait()
  plsc.subcore_barrier()
  for cp in pull_copies(0):
    cp.start()

  for c in range(_NCHUNK):
    s = c % 2
    for cp in pull_copies(s):
      cp.wait()
    if c + 1 < _NCHUNK:
      stage_copy(c + 1, 1 - s).wait()
    plsc.subcore_barrier()
    if c + 1 < _NCHUNK:
      for cp in pull_copies(1 - s):
        cp.start()
    if c + 2 < _NCHUNK:
      stage_copy(c + 2, s).start()
    if c >= 2:
      for cp in out_copies(c - 2, s):
        cp.wait()
    compute(s)
    for cp in out_copies(c, s):
      cp.start()

  for cp in out_copies(_NCHUNK - 2, 0):
    cp.wait()
  for cp in out_copies(_NCHUNK - 1, 1):
    cp.wait()


def kernel(x, duty_cycle):
  mesh = plsc.VectorSubcoreMesh(core_axis_name="c", subcore_axis_name="s")
  scratch = [
      pltpu.VMEM((_WPAD,), jnp.float32),
      pltpu.VMEM((_NBLK * 64,), jnp.float32),
      pltpu.VMEM((_R, _WPAD), jnp.float32),
      pltpu.VMEM((_R, _WPAD), jnp.float32),
      pltpu.VMEM((_R, _XLEN), jnp.float32),
      pltpu.VMEM((_R, _XLEN), jnp.float32),
      pltpu.VMEM((_R, _XLEN), jnp.float32),
      pltpu.VMEM((_R, _XLEN), jnp.float32),
      pltpu.VMEM_SHARED((_R, _SHW), jnp.float32),
      pltpu.VMEM_SHARED((_R, _SHW), jnp.float32),
      pltpu.SemaphoreType.DMA,
      pltpu.SemaphoreType.DMA,
      pltpu.SemaphoreType.DMA,
      pltpu.SemaphoreType.DMA,
      pltpu.SemaphoreType.DMA,
      pltpu.SemaphoreType.DMA,
      pltpu.SemaphoreType.DMA,
      pltpu.SemaphoreType.DMA,
  ]
  run = pl.kernel(
      _dkw_body,
      out_type=jax.ShapeDtypeStruct((_B, _N), jnp.float32),
      mesh=mesh,
      scratch_types=scratch,
      compiler_params=pltpu.CompilerParams(
          needs_layout_passes=False, use_tc_tiling_on_sc=True),
  )
  return run(x, duty_cycle)


# speedup vs baseline: 1.0410x; 1.0410x over previous
import jax
import jax.numpy as jnp
from jax import lax
from jax.experimental import pallas as pl
from jax.experimental.pallas import tpu as pltpu
from jax.experimental.pallas import tpu_sc as plsc

_B = 128
_N = 32768
_OUT_DIM = 8192
_DPC = 4
_BOOST_STRENGTH = 1.0
_TARGET_DENSITY = float(_OUT_DIM) / _N

_NC = 2
_NS = 16
_NW = _NC * _NS
_GPW = _OUT_DIM // _NW
_WSPAN = (_DPC - 1) * _GPW
_WPAD = _WSPAN + 128
_XLEN = _DPC * _GPW
_R = 8
_NCHUNK = _B // _R
_NBLK = _GPW // 16


def _dkw_body(x_hbm, duty_hbm, out_hbm,
              dv, bf_tile, xw0, xw1, xo0, xo1, ob0, ob1,
              sem_w0, sem_w1, sem_o0, sem_o1, sem_out0, sem_out1):
  wid = lax.axis_index("s") * _NC + lax.axis_index("c")
  wcol = wid * _WSPAN
  xcol = wid * _XLEN

  iota = lax.iota(jnp.int32, 16)
  iota3 = iota * (_DPC - 1)

  xw = (xw0, xw1)
  xo = (xo0, xo1)
  ob = (ob0, ob1)
  sem_w = (sem_w0, sem_w1)
  sem_o = (sem_o0, sem_o1)
  sem_out = (sem_out0, sem_out1)

  def in_copies(c, s):
    rows = pl.ds(c * _R, _R)
    return [
        pltpu.make_async_copy(
            x_hbm.at[rows, pl.ds(wcol, _WPAD)], xw[s], sem_w[s]),
        pltpu.make_async_copy(
            x_hbm.at[rows, pl.ds(xcol, _XLEN)], xo[s], sem_o[s]),
    ]

  def out_copies(c, s):
    rows = pl.ds(c * _R, _R)
    return [pltpu.make_async_copy(
        ob[s], out_hbm.at[rows, pl.ds(xcol, _XLEN)], sem_out[s])]

  dnums = lax.GatherDimensionNumbers(
      offset_dims=(), collapsed_slice_dims=(0,), start_index_map=(0,))

  def vgather(v, idx):
    return lax.gather(v, idx[:, None], dnums, (1,),
                      mode=lax.GatherScatterMode.PROMISE_IN_BOUNDS)

  perm_base = iota // 4
  slot = iota - perm_base * 4

  def compute(s):
    xw_s, xo_s, ob_s = xw[s], xo[s], ob[s]

    @plsc.parallel_loop(0, _NBLK)
    def _(blk):
      cw = iota3 + blk * 48
      ob_off = blk * 64
      bfv = [bf_tile[pl.ds(blk * 64 + j * 16, 16)] for j in range(_DPC)]

      @plsc.parallel_loop(0, _R, unroll=4)
      def _(r):
        rvec = jnp.full((16,), r, jnp.int32)
        m = plsc.load_gather(xw_s, [rvec, cw]) * bfv[0]
        ind = jnp.zeros((16,), jnp.int32)
        for j in range(1, _DPC):
          wj = plsc.load_gather(xw_s, [rvec, cw + j]) * bfv[j]
          gt = wj > m
          m = jnp.where(gt, wj, m)
          ind = jnp.where(gt, j, ind)
        z = jnp.zeros((16,), jnp.float32)
        for j in range(_DPC):
          indp = vgather(ind, perm_base + 4 * j)
          xov = xo_s[r, pl.ds(ob_off + j * 16, 16)]
          ob_s[r, pl.ds(ob_off + j * 16, 16)] = jnp.where(indp == slot, xov, z)

  def bf_prologue():
    pltpu.sync_copy(duty_hbm.at[pl.ds(wcol, _WPAD)], dv)

    @pl.loop(0, _WPAD // 16)
    def _(i):
      v = dv[pl.ds(i * 16, 16)]
      dv[pl.ds(i * 16, 16)] = jnp.exp((_TARGET_DENSITY - v) * _BOOST_STRENGTH)

    @pl.loop(0, _NBLK)
    def _(blk):
      base = blk * 48
      for j in range(_DPC):
        vals = plsc.load_gather(dv, [iota3 + (base + j)])
        bf_tile[pl.ds(blk * 64 + j * 16, 16)] = vals

  for c in range(_NCHUNK):
    s = c % 2
    if c == 0:
      for cp in in_copies(0, 0):
        cp.start()
      for cp in in_copies(1, 1):
        cp.start()
      bf_prologue()
    if 0 < c and c + 1 < _NCHUNK:
      for cp in in_copies(c + 1, 1 - s):
        cp.start()
    for cp in in_copies(c, s):
      cp.wait()
    if c >= 2:
      for cp in out_copies(c - 2, s):
        cp.wait()
    compute(s)
    for cp in out_copies(c, s):
      cp.start()
  for cp in out_copies(_NCHUNK - 2, 0):
    cp.wait()
  for cp in out_copies(_NCHUNK - 1, 1):
    cp.wait()


def kernel(x, duty_cycle):
  mesh = plsc.VectorSubcoreMesh(core_axis_name="c", subcore_axis_name="s")
  scratch = [
      pltpu.VMEM((_WPAD,), jnp.float32),
      pltpu.VMEM((_NBLK * 64,), jnp.float32),
      pltpu.VMEM((_R, _WPAD), jnp.float32),
      pltpu.VMEM((_R, _WPAD), jnp.float32),
      pltpu.VMEM((_R, _XLEN), jnp.float32),
      pltpu.VMEM((_R, _XLEN), jnp.float32),
      pltpu.VMEM((_R, _XLEN), jnp.float32),
      pltpu.VMEM((_R, _XLEN), jnp.float32),
      pltpu.SemaphoreType.DMA,
      pltpu.SemaphoreType.DMA,
      pltpu.SemaphoreType.DMA,
      pltpu.SemaphoreType.DMA,
      pltpu.SemaphoreType.DMA,
      pltpu.SemaphoreType.DMA,
  ]
  run = pl.kernel(
      _dkw_body,
      out_type=jax.ShapeDtypeStruct((_B, _N), jnp.float32),
      mesh=mesh,
      scratch_types=scratch,
      compiler_params=pltpu.CompilerParams(
          needs_layout_passes=False, use_tc_tiling_on_sc=True),
  )
  return run(x, duty_cycle)


# speedup vs baseline: 1.0416x; 1.0006x over previous
import jax
import jax.numpy as jnp
from jax import lax
from jax.experimental import pallas as pl
from jax.experimental.pallas import tpu as pltpu
from jax.experimental.pallas import tpu_sc as plsc

_B = 128
_N = 32768
_OUT_DIM = 8192
_DPC = 4
_BOOST_STRENGTH = 1.0
_TARGET_DENSITY = float(_OUT_DIM) / _N

_NC = 2
_NS = 16
_NW = _NC * _NS
_GPW = _OUT_DIM // _NW
_WSPAN = (_DPC - 1) * _GPW
_WPAD = _WSPAN + 128
_XLEN = _DPC * _GPW
_R = 8
_NCHUNK = _B // _R
_NBLK = _GPW // 16


def _dkw_body(x_hbm, duty_hbm, out_hbm,
              dv, bf_tile, xw0, xw1, xo0, xo1, ob0, ob1,
              sem_w0, sem_w1, sem_o0, sem_o1, sem_out0, sem_out1):
  wid = lax.axis_index("s") * _NC + lax.axis_index("c")
  wcol = wid * _WSPAN
  xcol = wid * _XLEN

  iota = lax.iota(jnp.int32, 16)
  iota3 = iota * (_DPC - 1)

  xw = (xw0, xw1)
  xo = (xo0, xo1)
  ob = (ob0, ob1)
  sem_w = (sem_w0, sem_w1)
  sem_o = (sem_o0, sem_o1)
  sem_out = (sem_out0, sem_out1)

  def in_copies(c, s):
    rows = pl.ds(c * _R, _R)
    return [
        pltpu.make_async_copy(
            x_hbm.at[rows, pl.ds(wcol, _WPAD)], xw[s], sem_w[s]),
        pltpu.make_async_copy(
            x_hbm.at[rows, pl.ds(xcol, _XLEN)], xo[s], sem_o[s]),
    ]

  def out_copies(c, s):
    rows = pl.ds(c * _R, _R)
    return [pltpu.make_async_copy(
        ob[s], out_hbm.at[rows, pl.ds(xcol, _XLEN)], sem_out[s])]

  dnums = lax.GatherDimensionNumbers(
      offset_dims=(), collapsed_slice_dims=(0,), start_index_map=(0,))

  def vgather(v, idx):
    return lax.gather(v, idx[:, None], dnums, (1,),
                      mode=lax.GatherScatterMode.PROMISE_IN_BOUNDS)

  perm_base = iota // 4
  slot = iota - perm_base * 4

  def compute(s):
    xw_s, xo_s, ob_s = xw[s], xo[s], ob[s]

    @plsc.parallel_loop(0, _NBLK)
    def _(blk):
      cw = iota3 + blk * 48
      ob_off = blk * 64
      bfv = [bf_tile[pl.ds(blk * 64 + j * 16, 16)] for j in range(_DPC)]

      @plsc.parallel_loop(0, _R, unroll=2)
      def _(r):
        rvec = jnp.full((16,), r, jnp.int32)
        m = plsc.load_gather(xw_s, [rvec, cw]) * bfv[0]
        ind = jnp.zeros((16,), jnp.int32)
        for j in range(1, _DPC):
          wj = plsc.load_gather(xw_s, [rvec, cw + j]) * bfv[j]
          gt = wj > m
          m = jnp.where(gt, wj, m)
          ind = jnp.where(gt, j, ind)
        z = jnp.zeros((16,), jnp.float32)
        for j in range(_DPC):
          indp = vgather(ind, perm_base + 4 * j)
          xov = xo_s[r, pl.ds(ob_off + j * 16, 16)]
          ob_s[r, pl.ds(ob_off + j * 16, 16)] = jnp.where(indp == slot, xov, z)

  def bf_prologue():
    pltpu.sync_copy(duty_hbm.at[pl.ds(wcol, _WPAD)], dv)

    @pl.loop(0, _WPAD // 16)
    def _(i):
      v = dv[pl.ds(i * 16, 16)]
      dv[pl.ds(i * 16, 16)] = jnp.exp((_TARGET_DENSITY - v) * _BOOST_STRENGTH)

    @pl.loop(0, _NBLK)
    def _(blk):
      base = blk * 48
      for j in range(_DPC):
        vals = plsc.load_gather(dv, [iota3 + (base + j)])
        bf_tile[pl.ds(blk * 64 + j * 16, 16)] = vals

  for c in range(_NCHUNK):
    s = c % 2
    if c == 0:
      for cp in in_copies(0, 0):
        cp.start()
      for cp in in_copies(1, 1):
        cp.start()
      bf_prologue()
    if 0 < c and c + 1 < _NCHUNK:
      for cp in in_copies(c + 1, 1 - s):
        cp.start()
    for cp in in_copies(c, s):
      cp.wait()
    if c >= 2:
      for cp in out_copies(c - 2, s):
        cp.wait()
    compute(s)
    for cp in out_copies(c, s):
      cp.start()
  for cp in out_copies(_NCHUNK - 2, 0):
    cp.wait()
  for cp in out_copies(_NCHUNK - 1, 1):
    cp.wait()


def kernel(x, duty_cycle):
  mesh = plsc.VectorSubcoreMesh(core_axis_name="c", subcore_axis_name="s")
  scratch = [
      pltpu.VMEM((_WPAD,), jnp.float32),
      pltpu.VMEM((_NBLK * 64,), jnp.float32),
      pltpu.VMEM((_R, _WPAD), jnp.float32),
      pltpu.VMEM((_R, _WPAD), jnp.float32),
      pltpu.VMEM((_R, _XLEN), jnp.float32),
      pltpu.VMEM((_R, _XLEN), jnp.float32),
      pltpu.VMEM((_R, _XLEN), jnp.float32),
      pltpu.VMEM((_R, _XLEN), jnp.float32),
      pltpu.SemaphoreType.DMA,
      pltpu.SemaphoreType.DMA,
      pltpu.SemaphoreType.DMA,
      pltpu.SemaphoreType.DMA,
      pltpu.SemaphoreType.DMA,
      pltpu.SemaphoreType.DMA,
  ]
  run = pl.kernel(
      _dkw_body,
      out_type=jax.ShapeDtypeStruct((_B, _N), jnp.float32),
      mesh=mesh,
      scratch_types=scratch,
      compiler_params=pltpu.CompilerParams(
          needs_layout_passes=False, use_tc_tiling_on_sc=True),
  )
  return run(x, duty_cycle)


# speedup vs baseline: 1.1252x; 1.0804x over previous
import jax
import jax.numpy as jnp
from jax import lax
from jax.experimental import pallas as pl
from jax.experimental.pallas import tpu as pltpu
from jax.experimental.pallas import tpu_sc as plsc

_B = 128
_N = 32768
_OUT_DIM = 8192
_DPC = 4
_BOOST_STRENGTH = 1.0
_TARGET_DENSITY = float(_OUT_DIM) / _N

_NC = 2
_NS = 16
_NW = _NC * _NS
_GPW = _OUT_DIM // _NW
_WSPAN = (_DPC - 1) * _GPW
_WPAD = _WSPAN + 128
_XLEN = _DPC * _GPW
_R = 8
_NCHUNK = _B // _R
_NBLK = _GPW // 16


def _dkw_body(x_hbm, duty_hbm, out_hbm,
              dv, bf_tile, xw0, xw1, xo0, xo1, ob0, ob1,
              sem_w0, sem_w1, sem_o0, sem_o1, sem_out0, sem_out1):
  wid = lax.axis_index("s") * _NC + lax.axis_index("c")
  wcol = wid * _WSPAN
  xcol = wid * _XLEN

  iota = lax.iota(jnp.int32, 16)
  iota3 = iota * (_DPC - 1)

  xw = (xw0, xw1)
  xo = (xo0, xo1)
  ob = (ob0, ob1)
  sem_w = (sem_w0, sem_w1)
  sem_o = (sem_o0, sem_o1)
  sem_out = (sem_out0, sem_out1)

  def in_copies(c, s):
    rows = pl.ds(c * _R, _R)
    return [
        pltpu.make_async_copy(
            x_hbm.at[rows, pl.ds(wcol, _WPAD)], xw[s], sem_w[s]),
        pltpu.make_async_copy(
            x_hbm.at[rows, pl.ds(xcol, _XLEN)], xo[s], sem_o[s]),
    ]

  def out_copies(c, s):
    rows = pl.ds(c * _R, _R)
    return [pltpu.make_async_copy(
        ob[s], out_hbm.at[rows, pl.ds(xcol, _XLEN)], sem_out[s])]

  dnums = lax.GatherDimensionNumbers(
      offset_dims=(), collapsed_slice_dims=(0,), start_index_map=(0,))

  def vgather(v, idx):
    return lax.gather(v, idx[:, None], dnums, (1,),
                      mode=lax.GatherScatterMode.PROMISE_IN_BOUNDS)

  perm_base = iota // 4
  slot = iota - perm_base * 4

  def compute(s):
    xw_s, xo_s, ob_s = xw[s], xo[s], ob[s]

    @plsc.parallel_loop(0, _NBLK)
    def _(blk):
      cw = iota3 + blk * 48
      ob_off = blk * 64
      bfv = [bf_tile[pl.ds(blk * 64 + j * 16, 16)] for j in range(_DPC)]

      @plsc.parallel_loop(0, _R, unroll=2)
      def _(r):
        rvec = jnp.full((16,), r, jnp.int32)
        m = plsc.load_gather(xw_s, [rvec, cw]) * bfv[0]
        ind = jnp.zeros((16,), jnp.int32)
        for j in range(1, _DPC):
          wj = plsc.load_gather(xw_s, [rvec, cw + j]) * bfv[j]
          gt = wj > m
          m = jnp.where(gt, wj, m)
          ind = jnp.where(gt, j, ind)
        z = jnp.zeros((16,), jnp.float32)
        for j in range(_DPC):
          indp = vgather(ind, perm_base + 4 * j)
          xov = xo_s[r, pl.ds(ob_off + j * 16, 16)]
          ob_s[r, pl.ds(ob_off + j * 16, 16)] = jnp.where(indp == slot, xov, z)

  def bf_prologue():
    pltpu.sync_copy(duty_hbm.at[pl.ds(wcol, _WPAD)], dv)

    @pl.loop(0, _WPAD // 16)
    def _(i):
      v = dv[pl.ds(i * 16, 16)]
      dv[pl.ds(i * 16, 16)] = jnp.exp((_TARGET_DENSITY - v) * _BOOST_STRENGTH)

    @pl.loop(0, _NBLK)
    def _(blk):
      base = blk * 48
      for j in range(_DPC):
        vals = plsc.load_gather(dv, [iota3 + (base + j)])
        bf_tile[pl.ds(blk * 64 + j * 16, 16)] = vals

  for cp in in_copies(0, 0):
    cp.start()
  for cp in in_copies(1, 1):
    cp.start()
  bf_prologue()

  @pl.loop(0, _NCHUNK // 2)
  def _(p):
    a = 2 * p
    b = 2 * p + 1

    @pl.when(p > 0)
    def _():
      for cp in in_copies(b, 1):
        cp.start()
    for cp in in_copies(a, 0):
      cp.wait()

    @pl.when(p > 0)
    def _():
      for cp in out_copies(a - 2, 0):
        cp.wait()
    compute(0)
    for cp in out_copies(a, 0):
      cp.start()

    @pl.when(p < _NCHUNK // 2 - 1)
    def _():
      for cp in in_copies(b + 1, 0):
        cp.start()
    for cp in in_copies(b, 1):
      cp.wait()

    @pl.when(p > 0)
    def _():
      for cp in out_copies(b - 2, 1):
        cp.wait()
    compute(1)
    for cp in out_copies(b, 1):
      cp.start()

  for cp in out_copies(_NCHUNK - 2, 0):
    cp.wait()
  for cp in out_copies(_NCHUNK - 1, 1):
    cp.wait()


def kernel(x, duty_cycle):
  mesh = plsc.VectorSubcoreMesh(core_axis_name="c", subcore_axis_name="s")
  scratch = [
      pltpu.VMEM((_WPAD,), jnp.float32),
      pltpu.VMEM((_NBLK * 64,), jnp.float32),
      pltpu.VMEM((_R, _WPAD), jnp.float32),
      pltpu.VMEM((_R, _WPAD), jnp.float32),
      pltpu.VMEM((_R, _XLEN), jnp.float32),
      pltpu.VMEM((_R, _XLEN), jnp.float32),
      pltpu.VMEM((_R, _XLEN), jnp.float32),
      pltpu.VMEM((_R, _XLEN), jnp.float32),
      pltpu.SemaphoreType.DMA,
      pltpu.SemaphoreType.DMA,
      pltpu.SemaphoreType.DMA,
      pltpu.SemaphoreType.DMA,
      pltpu.SemaphoreType.DMA,
      pltpu.SemaphoreType.DMA,
  ]
  run = pl.kernel(
      _dkw_body,
      out_type=jax.ShapeDtypeStruct((_B, _N), jnp.float32),
      mesh=mesh,
      scratch_types=scratch,
      compiler_params=pltpu.CompilerParams(
          needs_layout_passes=False, use_tc_tiling_on_sc=True),
  )
  return run(x, duty_cycle)
